# Initial kernel scaffold; baseline (speedup 1.0000x reference)
import jax
import jax.numpy as jnp
from jax import lax
from jax.experimental import pallas as pl
from jax.experimental.pallas import tpu as pltpu
from jax.experimental.pallas import tpu_sc as plsc

N = 10000
E = 320000
NFEAT = 128
NHID = 256
HALF = 128
NCLASS = 16

NS = 16
NC = 2
NPAD = 10240
ROWS_PT = NPAD // NS

CHUNK = 128
EPT = 20096
NCHUNK = EPT // CHUNK
EPAD = EPT * NS

DEG_CHUNK = 64
DEG_EPT = EPAD // (NC * NS)
DEG_NCHUNK = DEG_EPT // DEG_CHUNK

_MESH = plsc.VectorSubcoreMesh(core_axis_name="c", subcore_axis_name="s")


def _deg_body(dst_hbm, deg_hbm, obuf, zbuf, idxb, acc):
    c = lax.axis_index("c")
    s = lax.axis_index("s")

    def fill_o(i, carry):
        obuf[i, :] = jnp.ones((16,), jnp.float32)
        return carry

    lax.fori_loop(0, DEG_CHUNK, fill_o, 0)

    def fill_z(i, carry):
        zbuf[i, :] = jnp.zeros((16,), jnp.float32)
        return carry

    lax.fori_loop(0, ROWS_PT, fill_z, 0)

    pltpu.sync_copy(zbuf, acc.at[pl.ds(s * ROWS_PT, ROWS_PT)])
    plsc.subcore_barrier()

    base = (c * NS + s) * DEG_EPT

    def step(j, carry):
        pltpu.sync_copy(dst_hbm.at[pl.ds(base + j * DEG_CHUNK, DEG_CHUNK)],
                        idxb.at[0])
        pltpu.sync_copy(obuf, acc.at[idxb.at[0]], add=True)
        return carry

    lax.fori_loop(0, DEG_NCHUNK, step, 0)
    plsc.subcore_barrier()
    pltpu.sync_copy(acc.at[pl.ds(s * ROWS_PT, ROWS_PT)],
                    deg_hbm.at[c, pl.ds(s * ROWS_PT, ROWS_PT)])


_deg_call = pl.kernel(
    _deg_body,
    out_type=jax.ShapeDtypeStruct((NC, NPAD, 16), jnp.float32),
    mesh=_MESH,
    scratch_types=[
        pltpu.VMEM((DEG_CHUNK, 16), jnp.float32),
        pltpu.VMEM((ROWS_PT, 16), jnp.float32),
        pltpu.VMEM((1, DEG_CHUNK), jnp.int32),
        pltpu.VMEM_SHARED((NPAD, 16), jnp.float32),
    ],
)


def _conv_body(g_hbm, src_hbm, dst_hbm, out_hbm, isrc, idst, gbuf, acc, sem):
    c = lax.axis_index("c")
    s = lax.axis_index("s")

    pltpu.sync_copy(g_hbm.at[pl.ds(c * NPAD + s * ROWS_PT, ROWS_PT)],
                    acc.at[pl.ds(s * ROWS_PT, ROWS_PT)])
    plsc.subcore_barrier()

    dbase = s * EPT
    sbase = c * EPAD + s * EPT

    def step(j, carry):
        pltpu.sync_copy(src_hbm.at[pl.ds(sbase + j * CHUNK, CHUNK)], isrc.at[0])
        pltpu.sync_copy(dst_hbm.at[pl.ds(dbase + j * CHUNK, CHUNK)], idst.at[0])
        pltpu.async_copy(g_hbm.at[isrc.at[0]], gbuf.at[0], sem).wait()
        pltpu.sync_copy(gbuf.at[0], acc.at[idst.at[0]], add=True)
        return carry

    lax.fori_loop(0, NCHUNK, step, 0)
    plsc.subcore_barrier()
    pltpu.sync_copy(acc.at[pl.ds(s * ROWS_PT, ROWS_PT)],
                    out_hbm.at[pl.ds(c * NPAD + s * ROWS_PT, ROWS_PT)])


_conv_call = pl.kernel(
    _conv_body,
    out_type=jax.ShapeDtypeStruct((NC * NPAD, HALF), jnp.float32),
    mesh=_MESH,
    scratch_types=[
        pltpu.VMEM((1, CHUNK), jnp.int32),
        pltpu.VMEM((1, CHUNK), jnp.int32),
        pltpu.VMEM((1, CHUNK, HALF), jnp.float32),
        pltpu.VMEM_SHARED((NPAD, HALF), jnp.float32),
        pltpu.SemaphoreType.DMA,
    ],
)


BN = 1024
GRID = NPAD // BN

_DN = (((1,), (1,)), ((), ()))


def _dis(degp):
    deg = degp[0, :, 0:1] + degp[1, :, 0:1] + 1.0
    return lax.rsqrt(deg)


def _k2_body(x_ref, degp_ref, w1_ref, out_ref):
    dis = _dis(degp_ref[...])
    h = lax.dot_general(x_ref[...], w1_ref[...], _DN,
                        preferred_element_type=jnp.float32)
    g = h * dis
    out_ref[0, :, :] = g[:, :HALF]
    out_ref[1, :, :] = g[:, HALF:]


_k2_call = pl.pallas_call(
    _k2_body,
    grid=(GRID,),
    in_specs=[
        pl.BlockSpec((BN, NFEAT), lambda i: (i, 0)),
        pl.BlockSpec((NC, BN, 16), lambda i: (0, i, 0)),
        pl.BlockSpec((NHID, NFEAT), lambda i: (0, 0)),
    ],
    out_specs=pl.BlockSpec((NC, BN, HALF), lambda i: (0, i, 0)),
    out_shape=jax.ShapeDtypeStruct((NC, NPAD, HALF), jnp.float32),
)


def _k4_body(acc_ref, degp_ref, w2_ref, b1_ref, out_ref):
    dis = _dis(degp_ref[...])
    a = jnp.concatenate([acc_ref[0], acc_ref[1]], axis=1)
    z = jax.nn.relu(a * dis + b1_ref[...])
    h = lax.dot_general(z, w2_ref[...], _DN, preferred_element_type=jnp.float32)
    g = h * dis
    out_ref[0, :, :] = g[:, :HALF]
    out_ref[1, :, :] = g[:, HALF:]


_k4_call = pl.pallas_call(
    _k4_body,
    grid=(GRID,),
    in_specs=[
        pl.BlockSpec((NC, BN, HALF), lambda i: (0, i, 0)),
        pl.BlockSpec((NC, BN, 16), lambda i: (0, i, 0)),
        pl.BlockSpec((NHID, NHID), lambda i: (0, 0)),
        pl.BlockSpec((1, NHID), lambda i: (0, 0)),
    ],
    out_specs=pl.BlockSpec((NC, BN, HALF), lambda i: (0, i, 0)),
    out_shape=jax.ShapeDtypeStruct((NC, NPAD, HALF), jnp.float32),
)


def _k6_body(acc_ref, degp_ref, b2_ref, wf1_ref, bf1_ref, wf2_ref, bf2_ref,
             out_ref):
    dis = _dis(degp_ref[...])
    a = jnp.concatenate([acc_ref[0], acc_ref[1]], axis=1)
    z2 = jax.nn.relu(a * dis + b2_ref[...])
    z3 = jax.nn.relu(lax.dot_general(z2, wf1_ref[...], _DN,
                                     preferred_element_type=jnp.float32)
                     + bf1_ref[...])
    o = lax.dot_general(z3, wf2_ref[...], _DN,
                        preferred_element_type=jnp.float32) + bf2_ref[...]
    out_ref[...] = o


_k6_call = pl.pallas_call(
    _k6_body,
    grid=(GRID,),
    in_specs=[
        pl.BlockSpec((NC, BN, HALF), lambda i: (0, i, 0)),
        pl.BlockSpec((NC, BN, 16), lambda i: (0, i, 0)),
        pl.BlockSpec((1, NHID), lambda i: (0, 0)),
        pl.BlockSpec((HALF, NHID), lambda i: (0, 0)),
        pl.BlockSpec((1, HALF), lambda i: (0, 0)),
        pl.BlockSpec((NCLASS, HALF), lambda i: (0, 0)),
        pl.BlockSpec((1, NCLASS), lambda i: (0, 0)),
    ],
    out_specs=pl.BlockSpec((BN, NCLASS), lambda i: (i, 0)),
    out_shape=jax.ShapeDtypeStruct((NPAD, NCLASS), jnp.float32),
)


def kernel(x, edge_index, W1, b1, W2, b2, Wf1, bf1, Wf2, bf2):
    x_pad = jnp.pad(x, ((0, NPAD - N), (0, 0)))
    src = edge_index[0]
    dst = edge_index[1]
    ept0 = E // NS
    srcp = jnp.pad(src.reshape(NS, ept0), ((0, 0), (0, EPT - ept0))).reshape(-1)
    dstp = jnp.pad(dst.reshape(NS, ept0), ((0, 0), (0, EPT - ept0)),
                   constant_values=N).reshape(-1)
    src2 = jnp.concatenate([srcp, srcp + NPAD])

    degp = _deg_call(dstp)
    g1 = _k2_call(x_pad, degp, W1)
    acc1 = _conv_call(g1.reshape(NC * NPAD, HALF), src2, dstp)
    g2 = _k4_call(acc1.reshape(NC, NPAD, HALF), degp, W2, b1.reshape(1, -1))
    acc2 = _conv_call(g2.reshape(NC * NPAD, HALF), src2, dstp)
    out = _k6_call(acc2.reshape(NC, NPAD, HALF), degp, b2.reshape(1, -1),
                   Wf1, bf1.reshape(1, -1), Wf2, bf2.reshape(1, -1))
    return out[:N]


# speedup vs baseline: 8.0977x; 8.0977x over previous
import jax
import jax.numpy as jnp
from jax import lax
from jax.experimental import pallas as pl
from jax.experimental.pallas import tpu as pltpu
from jax.experimental.pallas import tpu_sc as plsc

N = 10000
E = 320000
NFEAT = 128
NHID = 256
HALF = 128
NCLASS = 16

NS = 16
NC = 2
NPAD = 10240
ROWS_PT = NPAD // NS

CHUNK = 128
EPT = 20096
NCHUNK = EPT // CHUNK
EPAD = EPT * NS

_MESH = plsc.VectorSubcoreMesh(core_axis_name="c", subcore_axis_name="s")


def _conv_body(g_hbm, src_hbm, dst_hbm, out_hbm, isrc, idst, gbuf, acc, sem):
    c = lax.axis_index("c")
    s = lax.axis_index("s")

    pltpu.sync_copy(g_hbm.at[pl.ds(c * NPAD + s * ROWS_PT, ROWS_PT)],
                    acc.at[pl.ds(s * ROWS_PT, ROWS_PT)])
    plsc.subcore_barrier()

    dbase = s * EPT
    sbase = c * EPAD + s * EPT

    def step(j, carry):
        pltpu.sync_copy(src_hbm.at[pl.ds(sbase + j * CHUNK, CHUNK)], isrc.at[0])
        pltpu.sync_copy(dst_hbm.at[pl.ds(dbase + j * CHUNK, CHUNK)], idst.at[0])
        pltpu.async_copy(g_hbm.at[isrc.at[0]], gbuf.at[0], sem).wait()
        pltpu.sync_copy(gbuf.at[0], acc.at[idst.at[0]], add=True)
        return carry

    lax.fori_loop(0, NCHUNK, step, 0)
    plsc.subcore_barrier()
    pltpu.sync_copy(acc.at[pl.ds(s * ROWS_PT, ROWS_PT)],
                    out_hbm.at[pl.ds(c * NPAD + s * ROWS_PT, ROWS_PT)])


_conv_call = pl.kernel(
    _conv_body,
    out_type=jax.ShapeDtypeStruct((NC * NPAD, HALF), jnp.float32),
    mesh=_MESH,
    scratch_types=[
        pltpu.VMEM((1, CHUNK), jnp.int32),
        pltpu.VMEM((1, CHUNK), jnp.int32),
        pltpu.VMEM((1, CHUNK, HALF), jnp.float32),
        pltpu.VMEM_SHARED((NPAD, HALF), jnp.float32),
        pltpu.SemaphoreType.DMA,
    ],
)


BN = 1024
GRID = NPAD // BN

_DN = (((1,), (1,)), ((), ()))


def _dis(degp):
    return lax.rsqrt(degp[:, 0:1])


def _k2_body(x_ref, degp_ref, w1_ref, out_ref):
    dis = _dis(degp_ref[...])
    h = lax.dot_general(x_ref[...], w1_ref[...], _DN,
                        preferred_element_type=jnp.float32)
    g = h * dis
    out_ref[0, :, :] = g[:, :HALF]
    out_ref[1, :, :] = g[:, HALF:]


_k2_call = pl.pallas_call(
    _k2_body,
    grid=(GRID,),
    in_specs=[
        pl.BlockSpec((BN, NFEAT), lambda i: (i, 0)),
        pl.BlockSpec((BN, HALF), lambda i: (i, 0)),
        pl.BlockSpec((NHID, NFEAT), lambda i: (0, 0)),
    ],
    out_specs=pl.BlockSpec((NC, BN, HALF), lambda i: (0, i, 0)),
    out_shape=jax.ShapeDtypeStruct((NC, NPAD, HALF), jnp.float32),
)


def _k4_body(acc_ref, degp_ref, w2_ref, b1_ref, out_ref):
    dis = _dis(degp_ref[...])
    a = jnp.concatenate([acc_ref[0], acc_ref[1]], axis=1)
    z = jax.nn.relu(a * dis + b1_ref[...])
    h = lax.dot_general(z, w2_ref[...], _DN, preferred_element_type=jnp.float32)
    g = h * dis
    out_ref[0, :, :] = g[:, :HALF]
    out_ref[1, :, :] = g[:, HALF:]


_k4_call = pl.pallas_call(
    _k4_body,
    grid=(GRID,),
    in_specs=[
        pl.BlockSpec((NC, BN, HALF), lambda i: (0, i, 0)),
        pl.BlockSpec((BN, HALF), lambda i: (i, 0)),
        pl.BlockSpec((NHID, NHID), lambda i: (0, 0)),
        pl.BlockSpec((1, NHID), lambda i: (0, 0)),
    ],
    out_specs=pl.BlockSpec((NC, BN, HALF), lambda i: (0, i, 0)),
    out_shape=jax.ShapeDtypeStruct((NC, NPAD, HALF), jnp.float32),
)


def _k6_body(acc_ref, degp_ref, b2_ref, wf1_ref, bf1_ref, wf2_ref, bf2_ref,
             out_ref):
    dis = _dis(degp_ref[...])
    a = jnp.concatenate([acc_ref[0], acc_ref[1]], axis=1)
    z2 = jax.nn.relu(a * dis + b2_ref[...])
    z3 = jax.nn.relu(lax.dot_general(z2, wf1_ref[...], _DN,
                                     preferred_element_type=jnp.float32)
                     + bf1_ref[...])
    o = lax.dot_general(z3, wf2_ref[...], _DN,
                        preferred_element_type=jnp.float32) + bf2_ref[...]
    out_ref[...] = o


_k6_call = pl.pallas_call(
    _k6_body,
    grid=(GRID,),
    in_specs=[
        pl.BlockSpec((NC, BN, HALF), lambda i: (0, i, 0)),
        pl.BlockSpec((BN, HALF), lambda i: (i, 0)),
        pl.BlockSpec((1, NHID), lambda i: (0, 0)),
        pl.BlockSpec((HALF, NHID), lambda i: (0, 0)),
        pl.BlockSpec((1, HALF), lambda i: (0, 0)),
        pl.BlockSpec((NCLASS, HALF), lambda i: (0, 0)),
        pl.BlockSpec((1, NCLASS), lambda i: (0, 0)),
    ],
    out_specs=pl.BlockSpec((BN, NCLASS), lambda i: (i, 0)),
    out_shape=jax.ShapeDtypeStruct((NPAD, NCLASS), jnp.float32),
)


def kernel(x, edge_index, W1, b1, W2, b2, Wf1, bf1, Wf2, bf2):
    x_pad = jnp.pad(x, ((0, NPAD - N), (0, 0)))
    src = edge_index[0]
    dst = edge_index[1]
    ept0 = E // NS
    npad_e = EPT - ept0
    src_fill = (jnp.arange(npad_e, dtype=src.dtype) * 97) % N
    dst_fill = N + (jnp.arange(npad_e, dtype=dst.dtype) % (NPAD - N))
    srcp = jnp.concatenate(
        [src.reshape(NS, ept0),
         jnp.broadcast_to(src_fill, (NS, npad_e))], axis=1).reshape(-1)
    dstp = jnp.concatenate(
        [dst.reshape(NS, ept0),
         jnp.broadcast_to(dst_fill, (NS, npad_e))], axis=1).reshape(-1)
    src2 = jnp.concatenate([srcp, srcp + NPAD])

    ones = jnp.ones((NC * NPAD, HALF), jnp.float32)
    degp = _conv_call(ones, src2, dstp)[:NPAD]
    g1 = _k2_call(x_pad, degp, W1)
    acc1 = _conv_call(g1.reshape(NC * NPAD, HALF), src2, dstp)
    g2 = _k4_call(acc1.reshape(NC, NPAD, HALF), degp, W2, b1.reshape(1, -1))
    acc2 = _conv_call(g2.reshape(NC * NPAD, HALF), src2, dstp)
    out = _k6_call(acc2.reshape(NC, NPAD, HALF), degp, b2.reshape(1, -1),
                   Wf1, bf1.reshape(1, -1), Wf2, bf2.reshape(1, -1))
    return out[:N]


# speedup vs baseline: 13.6078x; 1.6804x over previous
import jax
import jax.numpy as jnp
from jax import lax
from jax.experimental import pallas as pl
from jax.experimental.pallas import tpu as pltpu
from jax.experimental.pallas import tpu_sc as plsc

N = 10000
E = 320000
NFEAT = 128
NHID = 256
HALF = 128
NCLASS = 16

NS = 16
NC = 2
NPAD = 10240
ROWS_PT = NPAD // NS

CHUNK = 128
EPT = 20480
NCHUNK = EPT // CHUNK
EPAD = EPT * NS
NBUF = 2
NGRP = 2

_MESH = plsc.VectorSubcoreMesh(core_axis_name="c", subcore_axis_name="s")


def _conv_body(g_hbm, src_hbm, dst_hbm, out_hbm, isrc, idst, gbuf, acc, sems):
    c = lax.axis_index("c")
    s = lax.axis_index("s")

    pltpu.sync_copy(g_hbm.at[pl.ds(c * NPAD + s * ROWS_PT, ROWS_PT)],
                    acc.at[pl.ds(s * ROWS_PT, ROWS_PT)])
    plsc.subcore_barrier()

    srow = c * (EPAD // CHUNK) + s * NCHUNK
    drow = s * NCHUNK

    pltpu.sync_copy(src_hbm.at[pl.ds(srow, NBUF)], isrc.at[0])
    pltpu.sync_copy(dst_hbm.at[pl.ds(drow, NBUF)], idst.at[0])
    for b in range(NBUF - 1):
        pltpu.async_copy(g_hbm.at[isrc.at[0, b]], gbuf.at[b], sems.at[b])

    @pl.loop(0, NCHUNK, step=NBUF)
    def _loop(j):
        gcur = lax.rem(j // NBUF, NGRP)
        gnext = lax.rem(j // NBUF + 1, NGRP)
        for b in range(NBUF):
            p = j + b + (NBUF - 1)
            slotp = (b + NBUF - 1) % NBUF

            @pl.when(p < NCHUNK)
            def _(b=b, p=p, slotp=slotp, gcur=gcur, gnext=gnext):
                if b == 0:
                    pltpu.async_copy(g_hbm.at[isrc.at[gcur, slotp]],
                                     gbuf.at[slotp], sems.at[slotp])
                else:
                    if b == 1:
                        pltpu.sync_copy(
                            src_hbm.at[pl.ds(srow + j + NBUF, NBUF)],
                            isrc.at[gnext])
                        pltpu.sync_copy(
                            dst_hbm.at[pl.ds(drow + j + NBUF, NBUF)],
                            idst.at[gnext])
                    pltpu.async_copy(g_hbm.at[isrc.at[gnext, slotp]],
                                     gbuf.at[slotp], sems.at[slotp])

            pltpu.make_async_copy(g_hbm.at[isrc.at[gcur, b]], gbuf.at[b],
                                  sems.at[b]).wait()
            pltpu.sync_copy(gbuf.at[b], acc.at[idst.at[gcur, b]], add=True)

    plsc.subcore_barrier()
    pltpu.sync_copy(acc.at[pl.ds(s * ROWS_PT, ROWS_PT)],
                    out_hbm.at[pl.ds(c * NPAD + s * ROWS_PT, ROWS_PT)])


_conv_call = pl.kernel(
    _conv_body,
    out_type=jax.ShapeDtypeStruct((NC * NPAD, HALF), jnp.float32),
    mesh=_MESH,
    scratch_types=[
        pltpu.VMEM((NGRP, NBUF, CHUNK), jnp.int32),
        pltpu.VMEM((NGRP, NBUF, CHUNK), jnp.int32),
        pltpu.VMEM((NBUF, CHUNK, HALF), jnp.float32),
        pltpu.VMEM_SHARED((NPAD, HALF), jnp.float32),
        pltpu.SemaphoreType.DMA((NBUF,)),
    ],
)


BN = 1024
GRID = NPAD // BN

_DN = (((1,), (1,)), ((), ()))


def _dis(degp):
    return lax.rsqrt(degp[:, 0:1])


def _k2_body(x_ref, degp_ref, w1_ref, out_ref):
    dis = _dis(degp_ref[...])
    h = lax.dot_general(x_ref[...], w1_ref[...], _DN,
                        preferred_element_type=jnp.float32)
    g = h * dis
    out_ref[0, :, :] = g[:, :HALF]
    out_ref[1, :, :] = g[:, HALF:]


_k2_call = pl.pallas_call(
    _k2_body,
    grid=(GRID,),
    in_specs=[
        pl.BlockSpec((BN, NFEAT), lambda i: (i, 0)),
        pl.BlockSpec((BN, HALF), lambda i: (i, 0)),
        pl.BlockSpec((NHID, NFEAT), lambda i: (0, 0)),
    ],
    out_specs=pl.BlockSpec((NC, BN, HALF), lambda i: (0, i, 0)),
    out_shape=jax.ShapeDtypeStruct((NC, NPAD, HALF), jnp.float32),
)


def _k4_body(acc_ref, degp_ref, w2_ref, b1_ref, out_ref):
    dis = _dis(degp_ref[...])
    a = jnp.concatenate([acc_ref[0], acc_ref[1]], axis=1)
    z = jax.nn.relu(a * dis + b1_ref[...])
    h = lax.dot_general(z, w2_ref[...], _DN, preferred_element_type=jnp.float32)
    g = h * dis
    out_ref[0, :, :] = g[:, :HALF]
    out_ref[1, :, :] = g[:, HALF:]


_k4_call = pl.pallas_call(
    _k4_body,
    grid=(GRID,),
    in_specs=[
        pl.BlockSpec((NC, BN, HALF), lambda i: (0, i, 0)),
        pl.BlockSpec((BN, HALF), lambda i: (i, 0)),
        pl.BlockSpec((NHID, NHID), lambda i: (0, 0)),
        pl.BlockSpec((1, NHID), lambda i: (0, 0)),
    ],
    out_specs=pl.BlockSpec((NC, BN, HALF), lambda i: (0, i, 0)),
    out_shape=jax.ShapeDtypeStruct((NC, NPAD, HALF), jnp.float32),
)


def _k6_body(acc_ref, degp_ref, b2_ref, wf1_ref, bf1_ref, wf2_ref, bf2_ref,
             out_ref):
    dis = _dis(degp_ref[...])
    a = jnp.concatenate([acc_ref[0], acc_ref[1]], axis=1)
    z2 = jax.nn.relu(a * dis + b2_ref[...])
    z3 = jax.nn.relu(lax.dot_general(z2, wf1_ref[...], _DN,
                                     preferred_element_type=jnp.float32)
                     + bf1_ref[...])
    o = lax.dot_general(z3, wf2_ref[...], _DN,
                        preferred_element_type=jnp.float32) + bf2_ref[...]
    out_ref[...] = o


_k6_call = pl.pallas_call(
    _k6_body,
    grid=(GRID,),
    in_specs=[
        pl.BlockSpec((NC, BN, HALF), lambda i: (0, i, 0)),
        pl.BlockSpec((BN, HALF), lambda i: (i, 0)),
        pl.BlockSpec((1, NHID), lambda i: (0, 0)),
        pl.BlockSpec((HALF, NHID), lambda i: (0, 0)),
        pl.BlockSpec((1, HALF), lambda i: (0, 0)),
        pl.BlockSpec((NCLASS, HALF), lambda i: (0, 0)),
        pl.BlockSpec((1, NCLASS), lambda i: (0, 0)),
    ],
    out_specs=pl.BlockSpec((BN, NCLASS), lambda i: (i, 0)),
    out_shape=jax.ShapeDtypeStruct((NPAD, NCLASS), jnp.float32),
)


def kernel(x, edge_index, W1, b1, W2, b2, Wf1, bf1, Wf2, bf2):
    x_pad = jnp.pad(x, ((0, NPAD - N), (0, 0)))
    src = edge_index[0]
    dst = edge_index[1]
    ept0 = E // NS
    npad_e = EPT - ept0
    src_fill = (jnp.arange(npad_e, dtype=src.dtype) * 97) % N
    dst_fill = N + (jnp.arange(npad_e, dtype=dst.dtype) % (NPAD - N))
    srcp = jnp.concatenate(
        [src.reshape(NS, ept0),
         jnp.broadcast_to(src_fill, (NS, npad_e))], axis=1).reshape(-1)
    dstp = jnp.concatenate(
        [dst.reshape(NS, ept0),
         jnp.broadcast_to(dst_fill, (NS, npad_e))], axis=1).reshape(-1)
    src2 = jnp.concatenate([srcp, srcp + NPAD]).reshape(-1, CHUNK)
    dstp = dstp.reshape(-1, CHUNK)

    ones = jnp.ones((NC * NPAD, HALF), jnp.float32)
    degp = _conv_call(ones, src2, dstp)[:NPAD]
    g1 = _k2_call(x_pad, degp, W1)
    acc1 = _conv_call(g1.reshape(NC * NPAD, HALF), src2, dstp)
    g2 = _k4_call(acc1.reshape(NC, NPAD, HALF), degp, W2, b1.reshape(1, -1))
    acc2 = _conv_call(g2.reshape(NC * NPAD, HALF), src2, dstp)
    out = _k6_call(acc2.reshape(NC, NPAD, HALF), degp, b2.reshape(1, -1),
                   Wf1, bf1.reshape(1, -1), Wf2, bf2.reshape(1, -1))
    return out[:N]


# speedup vs baseline: 17.7837x; 1.3069x over previous
import jax
import jax.numpy as jnp
from jax import lax
from jax.experimental import pallas as pl
from jax.experimental.pallas import tpu as pltpu
from jax.experimental.pallas import tpu_sc as plsc

N = 10000
E = 320000
NFEAT = 128
NHID = 256
HALF = 128
NCLASS = 16

NS = 16
NC = 2
NPAD = 10240
ROWS_PT = NPAD // NS

CHUNK = 128
EPT = 20480
NCHUNK = EPT // CHUNK
EPAD = EPT * NS
NBUF = 2
NGRP = 2

_MESH = plsc.VectorSubcoreMesh(core_axis_name="c", subcore_axis_name="s")


DCH = EPAD // CHUNK // (NC * NS)
DGRP = 4


def _deg_body(dst_hbm, deg_hbm, obuf, idxb, acc, sem):
    c = lax.axis_index("c")
    s = lax.axis_index("s")

    def fill_o(i, carry):
        def fo(j, carry2):
            obuf[i, pl.ds(j * 16, 16)] = jnp.ones((16,), jnp.float32)
            return carry2

        lax.fori_loop(0, HALF // 16, fo, 0)
        return carry

    lax.fori_loop(0, CHUNK, fill_o, 0)

    @pl.loop(0, ROWS_PT, step=CHUNK)
    def _init(r):
        pltpu.sync_copy(obuf, acc.at[pl.ds(s * ROWS_PT + r, CHUNK)])

    plsc.subcore_barrier()

    base = (c * NS + s) * DCH

    pltpu.sync_copy(dst_hbm.at[pl.ds(base, DGRP)], idxb.at[0])

    @pl.loop(0, DCH // DGRP)
    def _grp(g):
        slot = lax.rem(g, 2)
        slotn = lax.rem(g + 1, 2)
        for b in range(DGRP):
            pltpu.async_copy(obuf, acc.at[idxb.at[slot, b]], sem, add=True)

        @pl.when(g + 1 < DCH // DGRP)
        def _():
            pltpu.sync_copy(dst_hbm.at[pl.ds(base + (g + 1) * DGRP, DGRP)],
                            idxb.at[slotn])

        for b in range(DGRP):
            pltpu.make_async_copy(obuf, acc.at[idxb.at[slot, b]], sem).wait()

    plsc.subcore_barrier()
    pltpu.sync_copy(acc.at[pl.ds(s * ROWS_PT, ROWS_PT)],
                    deg_hbm.at[pl.ds(c * NPAD + s * ROWS_PT, ROWS_PT)])


_deg_call = pl.kernel(
    _deg_body,
    out_type=jax.ShapeDtypeStruct((NC * NPAD, HALF), jnp.float32),
    mesh=_MESH,
    scratch_types=[
        pltpu.VMEM((CHUNK, HALF), jnp.float32),
        pltpu.VMEM((2, DGRP, CHUNK), jnp.int32),
        pltpu.VMEM_SHARED((NPAD, HALF), jnp.float32),
        pltpu.SemaphoreType.DMA,
    ],
)


def _conv_body(g_hbm, src_hbm, dst_hbm, out_hbm, isrc, idst, gbuf, acc, sems):
    c = lax.axis_index("c")
    s = lax.axis_index("s")

    pltpu.sync_copy(g_hbm.at[pl.ds(c * NPAD + s * ROWS_PT, ROWS_PT)],
                    acc.at[pl.ds(s * ROWS_PT, ROWS_PT)])
    plsc.subcore_barrier()

    srow = c * (EPAD // CHUNK) + s * NCHUNK
    drow = s * NCHUNK

    pltpu.sync_copy(src_hbm.at[pl.ds(srow, NBUF)], isrc.at[0])
    pltpu.sync_copy(dst_hbm.at[pl.ds(drow, NBUF)], idst.at[0])
    for b in range(NBUF - 1):
        pltpu.async_copy(g_hbm.at[isrc.at[0, b]], gbuf.at[b], sems.at[b])

    @pl.loop(0, NCHUNK, step=NBUF)
    def _loop(j):
        gcur = lax.rem(j // NBUF, NGRP)
        gnext = lax.rem(j // NBUF + 1, NGRP)
        for b in range(NBUF):
            p = j + b + (NBUF - 1)
            slotp = (b + NBUF - 1) % NBUF

            @pl.when(p < NCHUNK)
            def _(b=b, p=p, slotp=slotp, gcur=gcur, gnext=gnext):
                if b == 0:
                    pltpu.async_copy(g_hbm.at[isrc.at[gcur, slotp]],
                                     gbuf.at[slotp], sems.at[slotp])
                else:
                    if b == 1:
                        pltpu.sync_copy(
                            src_hbm.at[pl.ds(srow + j + NBUF, NBUF)],
                            isrc.at[gnext])
                        pltpu.sync_copy(
                            dst_hbm.at[pl.ds(drow + j + NBUF, NBUF)],
                            idst.at[gnext])
                    pltpu.async_copy(g_hbm.at[isrc.at[gnext, slotp]],
                                     gbuf.at[slotp], sems.at[slotp])

            pltpu.make_async_copy(g_hbm.at[isrc.at[gcur, b]], gbuf.at[b],
                                  sems.at[b]).wait()
            pltpu.sync_copy(gbuf.at[b], acc.at[idst.at[gcur, b]], add=True)

    plsc.subcore_barrier()
    pltpu.sync_copy(acc.at[pl.ds(s * ROWS_PT, ROWS_PT)],
                    out_hbm.at[pl.ds(c * NPAD + s * ROWS_PT, ROWS_PT)])


_conv_call = pl.kernel(
    _conv_body,
    out_type=jax.ShapeDtypeStruct((NC * NPAD, HALF), jnp.float32),
    mesh=_MESH,
    scratch_types=[
        pltpu.VMEM((NGRP, NBUF, CHUNK), jnp.int32),
        pltpu.VMEM((NGRP, NBUF, CHUNK), jnp.int32),
        pltpu.VMEM((NBUF, CHUNK, HALF), jnp.float32),
        pltpu.VMEM_SHARED((NPAD, HALF), jnp.float32),
        pltpu.SemaphoreType.DMA((NBUF,)),
    ],
)


BN = 1024
GRID = NPAD // BN

_DN = (((1,), (1,)), ((), ()))


def _dis(degp):
    return lax.rsqrt(degp[0, :, 0:1] + degp[1, :, 0:1] - 1.0)


def _k2_body(x_ref, degp_ref, w1_ref, out_ref):
    dis = _dis(degp_ref[...])
    h = lax.dot_general(x_ref[...], w1_ref[...], _DN,
                        preferred_element_type=jnp.float32)
    g = h * dis
    out_ref[0, :, :] = g[:, :HALF]
    out_ref[1, :, :] = g[:, HALF:]


_k2_call = pl.pallas_call(
    _k2_body,
    grid=(GRID,),
    in_specs=[
        pl.BlockSpec((BN, NFEAT), lambda i: (i, 0)),
        pl.BlockSpec((NC, BN, HALF), lambda i: (0, i, 0)),
        pl.BlockSpec((NHID, NFEAT), lambda i: (0, 0)),
    ],
    out_specs=pl.BlockSpec((NC, BN, HALF), lambda i: (0, i, 0)),
    out_shape=jax.ShapeDtypeStruct((NC, NPAD, HALF), jnp.float32),
)


def _k4_body(acc_ref, degp_ref, w2_ref, b1_ref, out_ref):
    dis = _dis(degp_ref[...])
    a = jnp.concatenate([acc_ref[0], acc_ref[1]], axis=1)
    z = jax.nn.relu(a * dis + b1_ref[...])
    h = lax.dot_general(z, w2_ref[...], _DN, preferred_element_type=jnp.float32)
    g = h * dis
    out_ref[0, :, :] = g[:, :HALF]
    out_ref[1, :, :] = g[:, HALF:]


_k4_call = pl.pallas_call(
    _k4_body,
    grid=(GRID,),
    in_specs=[
        pl.BlockSpec((NC, BN, HALF), lambda i: (0, i, 0)),
        pl.BlockSpec((NC, BN, HALF), lambda i: (0, i, 0)),
        pl.BlockSpec((NHID, NHID), lambda i: (0, 0)),
        pl.BlockSpec((1, NHID), lambda i: (0, 0)),
    ],
    out_specs=pl.BlockSpec((NC, BN, HALF), lambda i: (0, i, 0)),
    out_shape=jax.ShapeDtypeStruct((NC, NPAD, HALF), jnp.float32),
)


def _k6_body(acc_ref, degp_ref, b2_ref, wf1_ref, bf1_ref, wf2_ref, bf2_ref,
             out_ref):
    dis = _dis(degp_ref[...])
    a = jnp.concatenate([acc_ref[0], acc_ref[1]], axis=1)
    z2 = jax.nn.relu(a * dis + b2_ref[...])
    z3 = jax.nn.relu(lax.dot_general(z2, wf1_ref[...], _DN,
                                     preferred_element_type=jnp.float32)
                     + bf1_ref[...])
    o = lax.dot_general(z3, wf2_ref[...], _DN,
                        preferred_element_type=jnp.float32) + bf2_ref[...]
    out_ref[...] = o


_k6_call = pl.pallas_call(
    _k6_body,
    grid=(GRID,),
    in_specs=[
        pl.BlockSpec((NC, BN, HALF), lambda i: (0, i, 0)),
        pl.BlockSpec((NC, BN, HALF), lambda i: (0, i, 0)),
        pl.BlockSpec((1, NHID), lambda i: (0, 0)),
        pl.BlockSpec((HALF, NHID), lambda i: (0, 0)),
        pl.BlockSpec((1, HALF), lambda i: (0, 0)),
        pl.BlockSpec((NCLASS, HALF), lambda i: (0, 0)),
        pl.BlockSpec((1, NCLASS), lambda i: (0, 0)),
    ],
    out_specs=pl.BlockSpec((BN, NCLASS), lambda i: (i, 0)),
    out_shape=jax.ShapeDtypeStruct((NPAD, NCLASS), jnp.float32),
)


def kernel(x, edge_index, W1, b1, W2, b2, Wf1, bf1, Wf2, bf2):
    x_pad = jnp.pad(x, ((0, NPAD - N), (0, 0)))
    src = edge_index[0]
    dst = edge_index[1]
    ept0 = E // NS
    npad_e = EPT - ept0
    src_fill = (jnp.arange(npad_e, dtype=src.dtype) * 97) % N
    dst_fill = N + (jnp.arange(npad_e, dtype=dst.dtype) % (NPAD - N))
    srcp = jnp.concatenate(
        [src.reshape(NS, ept0),
         jnp.broadcast_to(src_fill, (NS, npad_e))], axis=1).reshape(-1)
    dstp = jnp.concatenate(
        [dst.reshape(NS, ept0),
         jnp.broadcast_to(dst_fill, (NS, npad_e))], axis=1).reshape(-1)
    src2 = jnp.concatenate([srcp, srcp + NPAD]).reshape(-1, CHUNK)
    dstp = dstp.reshape(-1, CHUNK)

    degp = _deg_call(dstp).reshape(NC, NPAD, HALF)
    g1 = _k2_call(x_pad, degp, W1)
    acc1 = _conv_call(g1.reshape(NC * NPAD, HALF), src2, dstp)
    g2 = _k4_call(acc1.reshape(NC, NPAD, HALF), degp, W2, b1.reshape(1, -1))
    acc2 = _conv_call(g2.reshape(NC * NPAD, HALF), src2, dstp)
    out = _k6_call(acc2.reshape(NC, NPAD, HALF), degp, b2.reshape(1, -1),
                   Wf1, bf1.reshape(1, -1), Wf2, bf2.reshape(1, -1))
    return out[:N]


# speedup vs baseline: 20.1246x; 1.1316x over previous
import jax
import jax.numpy as jnp
from jax import lax
from jax.experimental import pallas as pl
from jax.experimental.pallas import tpu as pltpu
from jax.experimental.pallas import tpu_sc as plsc

N = 10000
E = 320000
NFEAT = 128
NHID = 256
HALF = 128
NCLASS = 16

NS = 16
NC = 2
NPAD = 10240
ROWS_PT = NPAD // NS

CHUNK = 128
EPT = 20480
NCHUNK = EPT // CHUNK
EPAD = EPT * NS
NBUF = 2
NGRP = 2

_MESH = plsc.VectorSubcoreMesh(core_axis_name="c", subcore_axis_name="s")


DCH = EPAD // CHUNK // (NC * NS)
DGRP = 4


def _deg_body(dst_hbm, deg_hbm, obuf, idxb, acc, sem):
    c = lax.axis_index("c")
    s = lax.axis_index("s")

    def fill_o(i, carry):
        def fo(j, carry2):
            obuf[i, pl.ds(j * 16, 16)] = jnp.ones((16,), jnp.float32)
            return carry2

        lax.fori_loop(0, HALF // 16, fo, 0)
        return carry

    lax.fori_loop(0, CHUNK, fill_o, 0)

    @pl.loop(0, ROWS_PT, step=CHUNK)
    def _init(r):
        pltpu.sync_copy(obuf, acc.at[pl.ds(s * ROWS_PT + r, CHUNK)])

    plsc.subcore_barrier()

    base = (c * NS + s) * DCH

    pltpu.sync_copy(dst_hbm.at[pl.ds(base, DGRP)], idxb.at[0])

    @pl.loop(0, DCH // DGRP)
    def _grp(g):
        slot = lax.rem(g, 2)
        slotn = lax.rem(g + 1, 2)
        for b in range(DGRP):
            pltpu.async_copy(obuf, acc.at[idxb.at[slot, b]], sem, add=True)

        @pl.when(g + 1 < DCH // DGRP)
        def _():
            pltpu.sync_copy(dst_hbm.at[pl.ds(base + (g + 1) * DGRP, DGRP)],
                            idxb.at[slotn])

        for b in range(DGRP):
            pltpu.make_async_copy(obuf, acc.at[idxb.at[slot, b]], sem).wait()

    plsc.subcore_barrier()
    pltpu.sync_copy(acc.at[pl.ds(s * ROWS_PT, ROWS_PT)],
                    deg_hbm.at[pl.ds(c * NPAD + s * ROWS_PT, ROWS_PT)])


_deg_call = pl.kernel(
    _deg_body,
    out_type=jax.ShapeDtypeStruct((NC * NPAD, HALF), jnp.float32),
    mesh=_MESH,
    scratch_types=[
        pltpu.VMEM((CHUNK, HALF), jnp.float32),
        pltpu.VMEM((2, DGRP, CHUNK), jnp.int32),
        pltpu.VMEM_SHARED((NPAD, HALF), jnp.float32),
        pltpu.SemaphoreType.DMA,
    ],
)


def _conv_body(g_hbm, src_hbm, dst_hbm, out_hbm, isrc, idst, gbuf, acc, sems,
               ssems):
    c = lax.axis_index("c")
    s = lax.axis_index("s")

    pltpu.sync_copy(g_hbm.at[pl.ds(c * NPAD + s * ROWS_PT, ROWS_PT)],
                    acc.at[pl.ds(s * ROWS_PT, ROWS_PT)])
    plsc.subcore_barrier()

    srow = c * (EPAD // CHUNK) + s * NCHUNK
    drow = s * NCHUNK

    pltpu.sync_copy(src_hbm.at[pl.ds(srow, NBUF)], isrc.at[0])
    pltpu.sync_copy(dst_hbm.at[pl.ds(drow, NBUF)], idst.at[0])
    for b in range(NBUF - 1):
        pltpu.async_copy(g_hbm.at[isrc.at[0, b]], gbuf.at[b], sems.at[b])

    @pl.loop(0, NCHUNK, step=NBUF)
    def _loop(j):
        gcur = lax.rem(j // NBUF, NGRP)
        gnext = lax.rem(j // NBUF + 1, NGRP)
        gprev = gnext
        for b in range(NBUF):
            p = j + b + (NBUF - 1)
            slotp = (b + NBUF - 1) % NBUF

            @pl.when(p < NCHUNK)
            def _(b=b, p=p, slotp=slotp, gcur=gcur, gnext=gnext, gprev=gprev):
                if b == 0:
                    @pl.when(j > 0)
                    def _():
                        pltpu.make_async_copy(
                            gbuf.at[slotp], acc.at[idst.at[gprev, slotp]],
                            ssems.at[slotp]).wait()
                    pltpu.async_copy(g_hbm.at[isrc.at[gcur, slotp]],
                                     gbuf.at[slotp], sems.at[slotp])
                else:
                    if b == 1:
                        pltpu.sync_copy(
                            src_hbm.at[pl.ds(srow + j + NBUF, NBUF)],
                            isrc.at[gnext])
                        pltpu.sync_copy(
                            dst_hbm.at[pl.ds(drow + j + NBUF, NBUF)],
                            idst.at[gnext])
                    pltpu.make_async_copy(
                        gbuf.at[slotp], acc.at[idst.at[gcur, slotp]],
                        ssems.at[slotp]).wait()
                    pltpu.async_copy(g_hbm.at[isrc.at[gnext, slotp]],
                                     gbuf.at[slotp], sems.at[slotp])

            pltpu.make_async_copy(g_hbm.at[isrc.at[gcur, b]], gbuf.at[b],
                                  sems.at[b]).wait()
            pltpu.async_copy(gbuf.at[b], acc.at[idst.at[gcur, b]],
                             ssems.at[b], add=True)

    for b in range(NBUF):
        pltpu.make_async_copy(
            gbuf.at[b], acc.at[idst.at[(NCHUNK // NBUF - 1) % NGRP, b]],
            ssems.at[b]).wait()
    plsc.subcore_barrier()
    pltpu.sync_copy(acc.at[pl.ds(s * ROWS_PT, ROWS_PT)],
                    out_hbm.at[pl.ds(c * NPAD + s * ROWS_PT, ROWS_PT)])


_conv_call = pl.kernel(
    _conv_body,
    out_type=jax.ShapeDtypeStruct((NC * NPAD, HALF), jnp.float32),
    mesh=_MESH,
    scratch_types=[
        pltpu.VMEM((NGRP, NBUF, CHUNK), jnp.int32),
        pltpu.VMEM((NGRP, NBUF, CHUNK), jnp.int32),
        pltpu.VMEM((NBUF, CHUNK, HALF), jnp.float32),
        pltpu.VMEM_SHARED((NPAD, HALF), jnp.float32),
        pltpu.SemaphoreType.DMA((NBUF,)),
        pltpu.SemaphoreType.DMA((NBUF,)),
    ],
)


BN = 1024
GRID = NPAD // BN

_DN = (((1,), (1,)), ((), ()))


def _dis(degp):
    return lax.rsqrt(degp[0, :, 0:1] + degp[1, :, 0:1] - 1.0)


def _k2_body(x_ref, degp_ref, w1_ref, out_ref):
    dis = _dis(degp_ref[...])
    h = lax.dot_general(x_ref[...], w1_ref[...], _DN,
                        preferred_element_type=jnp.float32)
    g = h * dis
    out_ref[0, :, :] = g[:, :HALF]
    out_ref[1, :, :] = g[:, HALF:]


_k2_call = pl.pallas_call(
    _k2_body,
    grid=(GRID,),
    in_specs=[
        pl.BlockSpec((BN, NFEAT), lambda i: (i, 0)),
        pl.BlockSpec((NC, BN, HALF), lambda i: (0, i, 0)),
        pl.BlockSpec((NHID, NFEAT), lambda i: (0, 0)),
    ],
    out_specs=pl.BlockSpec((NC, BN, HALF), lambda i: (0, i, 0)),
    out_shape=jax.ShapeDtypeStruct((NC, NPAD, HALF), jnp.float32),
)


def _k4_body(acc_ref, degp_ref, w2_ref, b1_ref, out_ref):
    dis = _dis(degp_ref[...])
    a = jnp.concatenate([acc_ref[0], acc_ref[1]], axis=1)
    z = jax.nn.relu(a * dis + b1_ref[...])
    h = lax.dot_general(z, w2_ref[...], _DN, preferred_element_type=jnp.float32)
    g = h * dis
    out_ref[0, :, :] = g[:, :HALF]
    out_ref[1, :, :] = g[:, HALF:]


_k4_call = pl.pallas_call(
    _k4_body,
    grid=(GRID,),
    in_specs=[
        pl.BlockSpec((NC, BN, HALF), lambda i: (0, i, 0)),
        pl.BlockSpec((NC, BN, HALF), lambda i: (0, i, 0)),
        pl.BlockSpec((NHID, NHID), lambda i: (0, 0)),
        pl.BlockSpec((1, NHID), lambda i: (0, 0)),
    ],
    out_specs=pl.BlockSpec((NC, BN, HALF), lambda i: (0, i, 0)),
    out_shape=jax.ShapeDtypeStruct((NC, NPAD, HALF), jnp.float32),
)


def _k6_body(acc_ref, degp_ref, b2_ref, wf1_ref, bf1_ref, wf2_ref, bf2_ref,
             out_ref):
    dis = _dis(degp_ref[...])
    a = jnp.concatenate([acc_ref[0], acc_ref[1]], axis=1)
    z2 = jax.nn.relu(a * dis + b2_ref[...])
    z3 = jax.nn.relu(lax.dot_general(z2, wf1_ref[...], _DN,
                                     preferred_element_type=jnp.float32)
                     + bf1_ref[...])
    o = lax.dot_general(z3, wf2_ref[...], _DN,
                        preferred_element_type=jnp.float32) + bf2_ref[...]
    out_ref[...] = o


_k6_call = pl.pallas_call(
    _k6_body,
    grid=(GRID,),
    in_specs=[
        pl.BlockSpec((NC, BN, HALF), lambda i: (0, i, 0)),
        pl.BlockSpec((NC, BN, HALF), lambda i: (0, i, 0)),
        pl.BlockSpec((1, NHID), lambda i: (0, 0)),
        pl.BlockSpec((HALF, NHID), lambda i: (0, 0)),
        pl.BlockSpec((1, HALF), lambda i: (0, 0)),
        pl.BlockSpec((NCLASS, HALF), lambda i: (0, 0)),
        pl.BlockSpec((1, NCLASS), lambda i: (0, 0)),
    ],
    out_specs=pl.BlockSpec((BN, NCLASS), lambda i: (i, 0)),
    out_shape=jax.ShapeDtypeStruct((NPAD, NCLASS), jnp.float32),
)


def kernel(x, edge_index, W1, b1, W2, b2, Wf1, bf1, Wf2, bf2):
    x_pad = jnp.pad(x, ((0, NPAD - N), (0, 0)))
    src = edge_index[0]
    dst = edge_index[1]
    ept0 = E // NS
    npad_e = EPT - ept0
    src_fill = (jnp.arange(npad_e, dtype=src.dtype) * 97) % N
    dst_fill = N + (jnp.arange(npad_e, dtype=dst.dtype) % (NPAD - N))
    srcp = jnp.concatenate(
        [src.reshape(NS, ept0),
         jnp.broadcast_to(src_fill, (NS, npad_e))], axis=1).reshape(-1)
    dstp = jnp.concatenate(
        [dst.reshape(NS, ept0),
         jnp.broadcast_to(dst_fill, (NS, npad_e))], axis=1).reshape(-1)
    src2 = jnp.concatenate([srcp, srcp + NPAD]).reshape(-1, CHUNK)
    dstp = dstp.reshape(-1, CHUNK)

    degp = _deg_call(dstp).reshape(NC, NPAD, HALF)
    g1 = _k2_call(x_pad, degp, W1)
    acc1 = _conv_call(g1.reshape(NC * NPAD, HALF), src2, dstp)
    g2 = _k4_call(acc1.reshape(NC, NPAD, HALF), degp, W2, b1.reshape(1, -1))
    acc2 = _conv_call(g2.reshape(NC * NPAD, HALF), src2, dstp)
    out = _k6_call(acc2.reshape(NC, NPAD, HALF), degp, b2.reshape(1, -1),
                   Wf1, bf1.reshape(1, -1), Wf2, bf2.reshape(1, -1))
    return out[:N]
